# baseline (speedup 1.0000x reference)
import functools

import jax
import jax.numpy as jnp
from jax import lax
from jax.experimental import pallas as pl
from jax.experimental.pallas import tpu as pltpu
from jax.experimental.pallas import tpu_sc as plsc

_info = plsc.get_sparse_core_info()
_NC, _NS = _info.num_cores, _info.num_subcores
_NW = _NC * _NS
_CH = 16


@functools.lru_cache(maxsize=None)
def _make_gather(batch: int, dim: int):
    b_per_w = batch // _NW
    n_chunks = b_per_w // _CH
    half = n_chunks // 2
    mesh = plsc.VectorSubcoreMesh(core_axis_name="c", subcore_axis_name="s")

    @functools.partial(
        pl.kernel,
        mesh=mesh,
        out_type=jax.ShapeDtypeStruct((_NW, b_per_w, dim), jnp.float32),
        scratch_types=[
            pltpu.VMEM((n_chunks, _CH), jnp.int32),
            pltpu.VMEM((b_per_w,), jnp.int32),
            pltpu.VMEM((2, _CH, 2 * dim), jnp.float32),
            pltpu.VMEM((b_per_w, dim), jnp.float32),
            pltpu.SemaphoreType.DMA,
            pltpu.SemaphoreType.DMA,
        ],
        compiler_params=pltpu.CompilerParams(
            use_tc_tiling_on_sc=False, needs_layout_passes=False
        ),
    )
    def gather_kernel(table_hbm, pidx_hbm, hidx_hbm, out_hbm,
                      pidx_v, hidx_v, pairs_v, rows_v, sem0, sem1):
        wid = lax.axis_index("s") * _NC + lax.axis_index("c")
        sems = (sem0, sem1)
        pltpu.sync_copy(pidx_hbm.at[wid], pidx_v)
        pltpu.sync_copy(hidx_hbm.at[wid], hidx_v)
        pltpu.async_copy(table_hbm.at[pidx_v.at[0]], pairs_v.at[0], sem0)
        pltpu.async_copy(table_hbm.at[pidx_v.at[1]], pairs_v.at[1], sem1)

        lanes = lax.iota(jnp.int32, 16)

        def extract(j, b):
            r_vec = hidx_v[pl.ds(j * _CH, 16)] * dim
            for c in range(dim):
                vals = plsc.load_gather(pairs_v.at[b], [lanes, r_vec + c])
                plsc.store_scatter(
                    rows_v, [j * _CH + lanes, jnp.full((16,), c, jnp.int32)], vals
                )

        def body(i, _):
            for b in range(2):
                j = 2 * i + b
                pltpu.make_async_copy(
                    table_hbm.at[pl.ds(0, _CH)], pairs_v.at[b], sems[b]
                ).wait()
                extract(j, b)

                @pl.when(i < half - 1)
                def _():
                    pltpu.async_copy(
                        table_hbm.at[pidx_v.at[j + 2]], pairs_v.at[b], sems[b]
                    )
            return ()

        lax.fori_loop(0, half, body, ())
        pltpu.sync_copy(rows_v, out_hbm.at[wid])

    return gather_kernel


def kernel(states, embedding):
    batch = states.shape[0]
    v, dim = embedding.shape
    idx = states.astype(jnp.int32).reshape(batch)
    t2 = embedding.reshape(v // 2, 2 * dim)
    pidx = (idx >> 1).reshape(_NW, batch // (_NW * _CH), _CH)
    hidx = (idx & 1).reshape(_NW, batch // _NW)
    out = _make_gather(batch, dim)(t2, pidx, hidx)
    return out.reshape(batch, dim)
